# Initial kernel scaffold; baseline (speedup 1.0000x reference)
import jax
import jax.numpy as jnp
from jax.experimental import pallas as pl


def kernel(x, edge_index, W_mlp, b_mlp, gamma, beta, moving_mean, moving_var, W1, b1, W2, b2):
    raise NotImplementedError("write your pallas kernel here")


# speedup vs baseline: 12.8002x; 12.8002x over previous
import functools

import jax
import jax.numpy as jnp
from jax import lax
from jax.experimental import pallas as pl
from jax.experimental.pallas import tpu as pltpu
from jax.experimental.pallas import tpu_sc as plsc

N = 10000
E = 320000
D = 128
C = 32

NC = 2
NS = 16
NW = NC * NS
EPW = E // NW
CH = 80
NCHUNK = EPW // CH
NPAD = 10240
RPT = NPAD // NS

ROWBLK = 1024


@functools.cache
def _get_sc_kernel():
    mesh = plsc.VectorSubcoreMesh(core_axis_name="c", subcore_axis_name="s")

    @functools.partial(
        pl.kernel,
        mesh=mesh,
        compiler_params=pltpu.CompilerParams(use_tc_tiling_on_sc=False),
        out_type=[
            jax.ShapeDtypeStruct((NC, NPAD, C), jnp.float32),
            jax.ShapeDtypeStruct((NC, NPAD), jnp.float32),
        ],
        scratch_types=[
            pltpu.VMEM((EPW,), jnp.int32),
            pltpu.VMEM((NCHUNK, CH), jnp.int32),
            pltpu.VMEM((CH, C), jnp.float32),
            pltpu.VMEM((CH,), jnp.float32),
            pltpu.VMEM((RPT, C), jnp.float32),
            pltpu.VMEM((RPT,), jnp.float32),
            pltpu.VMEM_SHARED((NPAD, C), jnp.float32),
            pltpu.VMEM_SHARED((NPAD,), jnp.float32),
            pltpu.SemaphoreType.DMA,
        ],
    )
    def _sc_edge_aggregate(q_hbm, src_hbm, dst_hbm, out_acc, out_deg,
                           src_v, dst_v, rows_v, ones_v, zrows, zvec,
                           acc_sh, deg_sh, sem):
        _sc_body(q_hbm, src_hbm, dst_hbm, out_acc, out_deg,
                 src_v, dst_v, rows_v, ones_v, zrows, zvec,
                 acc_sh, deg_sh, sem)

    return _sc_edge_aggregate


def _sc_body(q_hbm, src_hbm, dst_hbm, out_acc, out_deg,
             src_v, dst_v, rows_v, ones_v, zrows, zvec,
             acc_sh, deg_sh, sem):
    c = lax.axis_index("c")
    s = lax.axis_index("s")
    w = c * NS + s

    zero16 = jnp.zeros((16,), jnp.float32)
    one16 = jnp.ones((16,), jnp.float32)

    def zfill_rows(i, carry):
        zrows[i, pl.ds(0, 16)] = zero16
        zrows[i, pl.ds(16, 16)] = zero16
        return carry

    lax.fori_loop(0, RPT, zfill_rows, 0)

    def zfill_vec(i, carry):
        zvec[pl.ds(i * 16, 16)] = zero16
        return carry

    lax.fori_loop(0, RPT // 16, zfill_vec, 0)

    for i in range(CH // 16):
        ones_v[pl.ds(i * 16, 16)] = one16

    pltpu.sync_copy(zrows, acc_sh.at[pl.ds(s * RPT, RPT)])
    pltpu.sync_copy(zvec, deg_sh.at[pl.ds(s * RPT, RPT)])

    pltpu.sync_copy(src_hbm.at[w], src_v)
    pltpu.sync_copy(dst_hbm.at[w], dst_v)
    plsc.subcore_barrier()

    def chunk(j, carry):
        pltpu.async_copy(q_hbm.at[src_v.at[pl.ds(j * CH, CH)]], rows_v, sem).wait()
        pltpu.sync_copy(rows_v, acc_sh.at[dst_v.at[j]], add=True)
        pltpu.sync_copy(ones_v, deg_sh.at[dst_v.at[j]], add=True)
        return carry

    lax.fori_loop(0, NCHUNK, chunk, 0)

    plsc.subcore_barrier()

    pltpu.sync_copy(acc_sh.at[pl.ds(s * RPT, RPT)], zrows)
    pltpu.sync_copy(zrows, out_acc.at[c, pl.ds(s * RPT, RPT)])
    pltpu.sync_copy(deg_sh.at[pl.ds(s * RPT, RPT)], zvec)
    pltpu.sync_copy(zvec, out_deg.at[c, pl.ds(s * RPT, RPT)])


def _precompute_tables(x, wa, wb, bp):
    def body(x_ref, wa_ref, wb_ref, bp_ref, p_ref, q_ref):
        xb = x_ref[...]
        p_ref[...] = jnp.dot(xb, wa_ref[...],
                             preferred_element_type=jnp.float32) + bp_ref[...]
        q_ref[...] = jnp.dot(xb, wb_ref[...],
                             preferred_element_type=jnp.float32)
    return pl.pallas_call(
        body,
        grid=(pl.cdiv(N, ROWBLK),),
        in_specs=[
            pl.BlockSpec((ROWBLK, D), lambda i: (i, 0)),
            pl.BlockSpec((D, C), lambda i: (0, 0)),
            pl.BlockSpec((D, C), lambda i: (0, 0)),
            pl.BlockSpec((1, C), lambda i: (0, 0)),
        ],
        out_specs=[
            pl.BlockSpec((ROWBLK, C), lambda i: (i, 0)),
            pl.BlockSpec((ROWBLK, C), lambda i: (i, 0)),
        ],
        out_shape=[
            jax.ShapeDtypeStruct((N, C), jnp.float32),
            jax.ShapeDtypeStruct((N, C), jnp.float32),
        ],
    )(x, wa, wb, bp)


def _heads(p, acc, deg, w1, b1t, w2, b2):
    def body(p_ref, acc_ref, deg_ref, w1_ref, b1_ref, w2_ref, b2_ref, o_ref):
        degb = deg_ref[0] + deg_ref[1]
        g = degb * p_ref[...] + acc_ref[0] + acc_ref[1]
        u = jnp.maximum(
            jnp.dot(g, w1_ref[...], preferred_element_type=jnp.float32)
            + b1_ref[...], 0.0)
        z = (jnp.dot(u, w2_ref[...], preferred_element_type=jnp.float32)
             + b2_ref[...])
        o_ref[...] = jax.nn.sigmoid(z)
    return pl.pallas_call(
        body,
        grid=(pl.cdiv(N, ROWBLK),),
        in_specs=[
            pl.BlockSpec((ROWBLK, C), lambda i: (i, 0)),
            pl.BlockSpec((NC, ROWBLK, C), lambda i: (0, i, 0)),
            pl.BlockSpec((NC, ROWBLK, 1), lambda i: (0, i, 0)),
            pl.BlockSpec((C, 16), lambda i: (0, 0)),
            pl.BlockSpec((1, 16), lambda i: (0, 0)),
            pl.BlockSpec((16, 1), lambda i: (0, 0)),
            pl.BlockSpec((1, 1), lambda i: (0, 0)),
        ],
        out_specs=pl.BlockSpec((ROWBLK, 1), lambda i: (i, 0)),
        out_shape=jax.ShapeDtypeStruct((N, 1), jnp.float32),
    )(p, acc, deg, w1, b1t, w2, b2)


def kernel(x, edge_index, W_mlp, b_mlp, gamma, beta, moving_mean,
           moving_var, W1, b1, W2, b2):
    s = gamma / jnp.sqrt(moving_var + 1e-3)
    t = beta - s * moving_mean
    wt = W_mlp[:D]
    wb = W_mlp[D:]
    wa = (wt - wb) * s[None, :]
    wbs = wb * s[None, :]
    bp = (b_mlp * s).reshape(1, C)
    b1t = (b1 + t @ W1).reshape(1, 16)

    p, q = _precompute_tables(x, wa, wbs, bp)

    src = edge_index[0].reshape(NW, EPW)
    dst = edge_index[1].reshape(NW, NCHUNK, CH)
    acc, deg = _get_sc_kernel()(q, src, dst)

    return _heads(p, acc, deg.reshape(NC, NPAD, 1),
                  W1, b1t, W2, b2.reshape(1, 1))


# speedup vs baseline: 21.0377x; 1.6435x over previous
import functools

import jax
import jax.numpy as jnp
from jax import lax
from jax.experimental import pallas as pl
from jax.experimental.pallas import tpu as pltpu
from jax.experimental.pallas import tpu_sc as plsc

N = 10000
E = 320000
D = 128
C = 32

NC = 2
NS = 16
NW = NC * NS
EPW = E // NW
CH = 80
NCHUNK = EPW // CH
GRP = 5
NGRP = NCHUNK // GRP
NPAD = 10240
RPT = NPAD // NS

ROWBLK = 1024


@functools.cache
def _get_sc_kernel():
    mesh = plsc.VectorSubcoreMesh(core_axis_name="c", subcore_axis_name="s")

    @functools.partial(
        pl.kernel,
        mesh=mesh,
        compiler_params=pltpu.CompilerParams(use_tc_tiling_on_sc=False),
        out_type=[
            jax.ShapeDtypeStruct((NC, NPAD, C), jnp.float32),
            jax.ShapeDtypeStruct((NC, NPAD), jnp.float32),
        ],
        scratch_types=[
            pltpu.VMEM((EPW,), jnp.int32),
            pltpu.VMEM((NCHUNK, CH), jnp.int32),
            pltpu.VMEM((GRP, CH, C), jnp.float32),
            pltpu.VMEM((GRP, CH, C), jnp.float32),
            pltpu.VMEM((CH,), jnp.float32),
            pltpu.VMEM((RPT, C), jnp.float32),
            pltpu.VMEM((RPT,), jnp.float32),
            pltpu.VMEM_SHARED((NPAD, C), jnp.float32),
            pltpu.VMEM_SHARED((NPAD,), jnp.float32),
            pltpu.SemaphoreType.DMA,
            pltpu.SemaphoreType.DMA,
        ],
    )
    def _sc_edge_aggregate(q_hbm, src_hbm, dst_hbm, out_acc, out_deg,
                           src_v, dst_v, rows_a, rows_b, ones_v, zrows, zvec,
                           acc_sh, deg_sh, sem_a, sem_b):
        _sc_body(q_hbm, src_hbm, dst_hbm, out_acc, out_deg,
                 src_v, dst_v, rows_a, rows_b, ones_v, zrows, zvec,
                 acc_sh, deg_sh, sem_a, sem_b)

    return _sc_edge_aggregate


def _sc_body(q_hbm, src_hbm, dst_hbm, out_acc, out_deg,
             src_v, dst_v, rows_a, rows_b, ones_v, zrows, zvec,
             acc_sh, deg_sh, sem_a, sem_b):
    c = lax.axis_index("c")
    s = lax.axis_index("s")
    w = c * NS + s

    zero16 = jnp.zeros((16,), jnp.float32)
    one16 = jnp.ones((16,), jnp.float32)

    def zfill_rows(i, carry):
        zrows[i, pl.ds(0, 16)] = zero16
        zrows[i, pl.ds(16, 16)] = zero16
        return carry

    lax.fori_loop(0, RPT, zfill_rows, 0)

    def zfill_vec(i, carry):
        zvec[pl.ds(i * 16, 16)] = zero16
        return carry

    lax.fori_loop(0, RPT // 16, zfill_vec, 0)

    for i in range(CH // 16):
        ones_v[pl.ds(i * 16, 16)] = one16

    pltpu.sync_copy(zrows, acc_sh.at[pl.ds(s * RPT, RPT)])
    pltpu.sync_copy(zvec, deg_sh.at[pl.ds(s * RPT, RPT)])

    pltpu.sync_copy(src_hbm.at[w], src_v)
    pltpu.sync_copy(dst_hbm.at[w], dst_v)
    plsc.subcore_barrier()

    def fire(g, buf, sem):
        return [
            pltpu.async_copy(
                q_hbm.at[src_v.at[pl.ds(g * (GRP * CH) + i * CH, CH)]],
                buf.at[i], sem)
            for i in range(GRP)
        ]

    def drain(g, buf, copies):
        for cp in copies:
            cp.wait()
        for i in range(GRP):
            pltpu.sync_copy(buf.at[i], acc_sh.at[dst_v.at[g * GRP + i]],
                            add=True)
            pltpu.sync_copy(ones_v, deg_sh.at[dst_v.at[g * GRP + i]],
                            add=True)

    fire(0, rows_a, sem_a)

    def grp_pair(m, carry):
        g0 = m * 2
        cps_b = fire(g0 + 1, rows_b, sem_b)
        cps_a = [
            pltpu.make_async_copy(
                q_hbm.at[src_v.at[pl.ds(g0 * (GRP * CH) + i * CH, CH)]],
                rows_a.at[i], sem_a)
            for i in range(GRP)
        ]
        drain(g0, rows_a, cps_a)
        fire(g0 + 2, rows_a, sem_a)
        drain(g0 + 1, rows_b, cps_b)
        return carry

    lax.fori_loop(0, (NGRP - 1) // 2, grp_pair, 0)

    last = NGRP - 1
    cps = [
        pltpu.make_async_copy(
            q_hbm.at[src_v.at[pl.ds(last * (GRP * CH) + i * CH, CH)]],
            rows_a.at[i], sem_a)
        for i in range(GRP)
    ]
    drain(last, rows_a, cps)

    plsc.subcore_barrier()

    pltpu.sync_copy(acc_sh.at[pl.ds(s * RPT, RPT)], zrows)
    pltpu.sync_copy(zrows, out_acc.at[c, pl.ds(s * RPT, RPT)])
    pltpu.sync_copy(deg_sh.at[pl.ds(s * RPT, RPT)], zvec)
    pltpu.sync_copy(zvec, out_deg.at[c, pl.ds(s * RPT, RPT)])


def _precompute_tables(x, wa, wb, bp):
    def body(x_ref, wa_ref, wb_ref, bp_ref, p_ref, q_ref):
        xb = x_ref[...]
        p_ref[...] = jnp.dot(xb, wa_ref[...],
                             preferred_element_type=jnp.float32) + bp_ref[...]
        q_ref[...] = jnp.dot(xb, wb_ref[...],
                             preferred_element_type=jnp.float32)
    return pl.pallas_call(
        body,
        grid=(pl.cdiv(N, ROWBLK),),
        in_specs=[
            pl.BlockSpec((ROWBLK, D), lambda i: (i, 0)),
            pl.BlockSpec((D, C), lambda i: (0, 0)),
            pl.BlockSpec((D, C), lambda i: (0, 0)),
            pl.BlockSpec((1, C), lambda i: (0, 0)),
        ],
        out_specs=[
            pl.BlockSpec((ROWBLK, C), lambda i: (i, 0)),
            pl.BlockSpec((ROWBLK, C), lambda i: (i, 0)),
        ],
        out_shape=[
            jax.ShapeDtypeStruct((N, C), jnp.float32),
            jax.ShapeDtypeStruct((N, C), jnp.float32),
        ],
    )(x, wa, wb, bp)


def _heads(p, acc, deg, w1, b1t, w2, b2):
    def body(p_ref, acc_ref, deg_ref, w1_ref, b1_ref, w2_ref, b2_ref, o_ref):
        degb = deg_ref[0] + deg_ref[1]
        g = degb * p_ref[...] + acc_ref[0] + acc_ref[1]
        u = jnp.maximum(
            jnp.dot(g, w1_ref[...], preferred_element_type=jnp.float32)
            + b1_ref[...], 0.0)
        z = (jnp.dot(u, w2_ref[...], preferred_element_type=jnp.float32)
             + b2_ref[...])
        o_ref[...] = jax.nn.sigmoid(z)
    return pl.pallas_call(
        body,
        grid=(pl.cdiv(N, ROWBLK),),
        in_specs=[
            pl.BlockSpec((ROWBLK, C), lambda i: (i, 0)),
            pl.BlockSpec((NC, ROWBLK, C), lambda i: (0, i, 0)),
            pl.BlockSpec((NC, ROWBLK, 1), lambda i: (0, i, 0)),
            pl.BlockSpec((C, 16), lambda i: (0, 0)),
            pl.BlockSpec((1, 16), lambda i: (0, 0)),
            pl.BlockSpec((16, 1), lambda i: (0, 0)),
            pl.BlockSpec((1, 1), lambda i: (0, 0)),
        ],
        out_specs=pl.BlockSpec((ROWBLK, 1), lambda i: (i, 0)),
        out_shape=jax.ShapeDtypeStruct((N, 1), jnp.float32),
    )(p, acc, deg, w1, b1t, w2, b2)


def kernel(x, edge_index, W_mlp, b_mlp, gamma, beta, moving_mean,
           moving_var, W1, b1, W2, b2):
    s = gamma / jnp.sqrt(moving_var + 1e-3)
    t = beta - s * moving_mean
    wt = W_mlp[:D]
    wb = W_mlp[D:]
    wa = (wt - wb) * s[None, :]
    wbs = wb * s[None, :]
    bp = (b_mlp * s).reshape(1, C)
    b1t = (b1 + t @ W1).reshape(1, 16)

    p, q = _precompute_tables(x, wa, wbs, bp)

    src = edge_index[0].reshape(NW, EPW)
    dst = edge_index[1].reshape(NW, NCHUNK, CH)
    acc, deg = _get_sc_kernel()(q, src, dst)

    return _heads(p, acc, deg.reshape(NC, NPAD, 1),
                  W1, b1t, W2, b2.reshape(1, 1))


# speedup vs baseline: 21.6076x; 1.0271x over previous
import functools

import jax
import jax.numpy as jnp
from jax import lax
from jax.experimental import pallas as pl
from jax.experimental.pallas import tpu as pltpu
from jax.experimental.pallas import tpu_sc as plsc

N = 10000
E = 320000
D = 128
C = 32

NC = 2
NS = 16
NW = NC * NS
EPW = E // NW
CH = 80
NCHUNK = EPW // CH
GRP = 5
NGRP = NCHUNK // GRP
NPAD = 10240
RPT = NPAD // NS


@functools.cache
def _get_sc_kernel():
    mesh = plsc.VectorSubcoreMesh(core_axis_name="c", subcore_axis_name="s")

    @functools.partial(
        pl.kernel,
        mesh=mesh,
        compiler_params=pltpu.CompilerParams(use_tc_tiling_on_sc=False),
        out_type=[
            jax.ShapeDtypeStruct((NC, NPAD, C), jnp.float32),
            jax.ShapeDtypeStruct((NC, NPAD), jnp.float32),
        ],
        scratch_types=[
            pltpu.VMEM((EPW,), jnp.int32),
            pltpu.VMEM((EPW,), jnp.int32),
            pltpu.VMEM((NCHUNK, CH), jnp.int32),
            pltpu.VMEM((GRP, CH, C), jnp.float32),
            pltpu.VMEM((GRP, CH, C), jnp.float32),
            pltpu.VMEM((CH,), jnp.float32),
            pltpu.VMEM((RPT, C), jnp.float32),
            pltpu.VMEM((RPT,), jnp.float32),
            pltpu.VMEM_SHARED((NPAD, C), jnp.float32),
            pltpu.VMEM_SHARED((NPAD,), jnp.float32),
            pltpu.SemaphoreType.DMA,
            pltpu.SemaphoreType.DMA,
            pltpu.SemaphoreType.DMA,
            pltpu.SemaphoreType.DMA,
        ],
    )
    def _sc_edge_aggregate(ei_hbm, q_hbm, out_acc, out_deg,
                           src_v, dst_f, dst_v, rows_a, rows_b, ones_v,
                           zrows, zcol, acc_sh, deg_sh,
                           gsem_a, gsem_b, ssem_a, ssem_b):
        _sc_body(ei_hbm, q_hbm, out_acc, out_deg,
                 src_v, dst_f, dst_v, rows_a, rows_b, ones_v,
                 zrows, zcol, acc_sh, deg_sh,
                 gsem_a, gsem_b, ssem_a, ssem_b)

    return _sc_edge_aggregate


def _sc_body(ei_hbm, q_hbm, out_acc, out_deg,
             src_v, dst_f, dst_v, rows_a, rows_b, ones_v,
             zrows, zcol, acc_sh, deg_sh,
             gsem_a, gsem_b, ssem_a, ssem_b):
    c = lax.axis_index("c")
    s = lax.axis_index("s")
    w = c * NS + s

    zero16 = jnp.zeros((16,), jnp.float32)
    one16 = jnp.ones((16,), jnp.float32)

    pltpu.sync_copy(ei_hbm.at[0, w], src_v)
    pltpu.sync_copy(ei_hbm.at[1, w], dst_f)

    VPR = CH // 16

    def dfill(i, carry):
        j = i // VPR
        k = i % VPR
        dst_v[j, pl.ds(k * 16, 16)] = dst_f[pl.ds(i * 16, 16)]
        return carry

    lax.fori_loop(0, NCHUNK * (CH // 16), dfill, 0)

    def zfill_rows(i, carry):
        zrows[i, pl.ds(0, 16)] = zero16
        zrows[i, pl.ds(16, 16)] = zero16
        return carry

    lax.fori_loop(0, RPT, zfill_rows, 0)

    def zfill_col(i, carry):
        zcol[pl.ds(i * 16, 16)] = zero16
        return carry

    lax.fori_loop(0, RPT // 16, zfill_col, 0)

    for i in range(CH // 16):
        ones_v[pl.ds(i * 16, 16)] = one16

    pltpu.sync_copy(zrows, acc_sh.at[pl.ds(s * RPT, RPT)])
    pltpu.sync_copy(zcol, deg_sh.at[pl.ds(s * RPT, RPT)])
    plsc.subcore_barrier()

    def fire_gather(g, buf, sem):
        for i in range(GRP):
            pltpu.async_copy(
                q_hbm.at[src_v.at[pl.ds(g * (GRP * CH) + i * CH, CH)]],
                buf.at[i], sem)

    def drain_gather(g, buf, sem):
        for i in range(GRP):
            pltpu.make_async_copy(
                q_hbm.at[src_v.at[pl.ds(g * (GRP * CH) + i * CH, CH)]],
                buf.at[i], sem).wait()

    def fire_scatter(g, buf, sem):
        for i in range(GRP):
            pltpu.async_copy(buf.at[i], acc_sh.at[dst_v.at[g * GRP + i]],
                             sem, add=True)
            pltpu.async_copy(ones_v, deg_sh.at[dst_v.at[g * GRP + i]],
                             sem, add=True)

    def drain_scatter(g, buf, sem):
        for i in range(GRP):
            pltpu.make_async_copy(buf.at[i], acc_sh.at[dst_v.at[g * GRP + i]],
                                  sem).wait()
            pltpu.make_async_copy(ones_v, deg_sh.at[dst_v.at[g * GRP + i]],
                                  sem).wait()

    fire_gather(0, rows_a, gsem_a)

    def grp_pair(m, carry):
        g0 = m * 2
        fire_gather(g0 + 1, rows_b, gsem_b)
        drain_gather(g0, rows_a, gsem_a)
        fire_scatter(g0, rows_a, ssem_a)
        drain_gather(g0 + 1, rows_b, gsem_b)
        fire_scatter(g0 + 1, rows_b, ssem_b)
        drain_scatter(g0, rows_a, ssem_a)
        fire_gather(g0 + 2, rows_a, gsem_a)
        drain_scatter(g0 + 1, rows_b, ssem_b)
        return carry

    lax.fori_loop(0, (NGRP - 1) // 2, grp_pair, 0)

    last = NGRP - 1
    drain_gather(last, rows_a, gsem_a)
    fire_scatter(last, rows_a, ssem_a)
    drain_scatter(last, rows_a, ssem_a)

    plsc.subcore_barrier()

    pltpu.sync_copy(acc_sh.at[pl.ds(s * RPT, RPT)], zrows)
    pltpu.sync_copy(zrows, out_acc.at[c, pl.ds(s * RPT, RPT)])
    pltpu.sync_copy(deg_sh.at[pl.ds(s * RPT, RPT)], zcol)
    pltpu.sync_copy(zcol, out_deg.at[c, pl.ds(s * RPT, RPT)])


def _precompute_tables(x, w_mlp, b_mlp):
    def body(x_ref, w_ref, b_ref, p_ref, q_ref):
        xb = x_ref[...]
        wt16 = w_ref[0:D, :].astype(jnp.bfloat16)
        wb16 = w_ref[D:2 * D, :].astype(jnp.bfloat16).astype(jnp.float32)
        qv = jnp.dot(xb, wb16,
                     preferred_element_type=jnp.float32,
                     precision=jax.lax.Precision.HIGHEST)
        p_ref[...] = (jnp.dot(xb.astype(jnp.bfloat16), wt16,
                              preferred_element_type=jnp.float32)
                      - qv + b_ref[...])
        q_ref[...] = qv
    return pl.pallas_call(
        body,
        in_specs=[
            pl.BlockSpec((N, D), lambda: (0, 0)),
            pl.BlockSpec((2 * D, C), lambda: (0, 0)),
            pl.BlockSpec((1, C), lambda: (0, 0)),
        ],
        out_specs=[
            pl.BlockSpec((N, C), lambda: (0, 0)),
            pl.BlockSpec((N, C), lambda: (0, 0)),
        ],
        out_shape=[
            jax.ShapeDtypeStruct((N, C), jnp.float32),
            jax.ShapeDtypeStruct((N, C), jnp.float32),
        ],
    )(x, w_mlp, b_mlp)


def _heads(p, acc, deg, gamma, beta, moving_mean, moving_var, w1, b1, w2, b2):
    def body(p_ref, acc_ref, deg_ref, g_ref, be_ref, mm_ref, v_ref,
             w1_ref, b1_ref, w2_ref, b2_ref, o_ref):
        degb = deg_ref[0, 0:N] + deg_ref[1, 0:N]
        h = degb * p_ref[...] + acc_ref[0, 0:N] + acc_ref[1, 0:N]
        hb = (g_ref[...] * (h - mm_ref[...])
              / jnp.sqrt(v_ref[...] + 1e-3) + be_ref[...])
        u = jnp.maximum(
            jnp.dot(hb.astype(jnp.bfloat16), w1_ref[...].astype(jnp.bfloat16),
                    preferred_element_type=jnp.float32) + b1_ref[...], 0.0)
        z = (jnp.dot(u.astype(jnp.bfloat16), w2_ref[...].astype(jnp.bfloat16),
                     preferred_element_type=jnp.float32) + b2_ref[...])
        o_ref[...] = jax.nn.sigmoid(z)
    return pl.pallas_call(
        body,
        in_specs=[
            pl.BlockSpec((N, C), lambda: (0, 0)),
            pl.BlockSpec((NC, NPAD, C), lambda: (0, 0, 0)),
            pl.BlockSpec((NC, NPAD, 1), lambda: (0, 0, 0)),
            pl.BlockSpec((1, C), lambda: (0, 0)),
            pl.BlockSpec((1, C), lambda: (0, 0)),
            pl.BlockSpec((1, C), lambda: (0, 0)),
            pl.BlockSpec((1, C), lambda: (0, 0)),
            pl.BlockSpec((C, 16), lambda: (0, 0)),
            pl.BlockSpec((1, 16), lambda: (0, 0)),
            pl.BlockSpec((16, 1), lambda: (0, 0)),
            pl.BlockSpec((1, 1), lambda: (0, 0)),
        ],
        out_specs=pl.BlockSpec((N, 1), lambda: (0, 0)),
        out_shape=jax.ShapeDtypeStruct((N, 1), jnp.float32),
    )(p, acc, deg, gamma, beta, moving_mean, moving_var, w1, b1, w2, b2)


def kernel(x, edge_index, W_mlp, b_mlp, gamma, beta, moving_mean,
           moving_var, W1, b1, W2, b2):
    g2 = gamma.reshape(1, C)
    be2 = beta.reshape(1, C)
    mm2 = moving_mean.reshape(1, C)
    mv2 = moving_var.reshape(1, C)

    p, q = _precompute_tables(x, W_mlp, b_mlp.reshape(1, C))

    ei3 = edge_index.reshape(2, NW, EPW)
    acc, deg = _get_sc_kernel()(ei3, q)

    return _heads(p, acc, deg.reshape(NC, NPAD, 1), g2, be2, mm2, mv2,
                  W1, b1.reshape(1, 16), W2, b2.reshape(1, 1))


# speedup vs baseline: 22.8367x; 1.0569x over previous
import functools

import jax
import jax.numpy as jnp
from jax import lax
from jax.experimental import pallas as pl
from jax.experimental.pallas import tpu as pltpu
from jax.experimental.pallas import tpu_sc as plsc

N = 10000
E = 320000
D = 128
C = 32

NC = 2
NS = 16
NW = NC * NS
CH = 128
CPT = 78
MAIN = CPT * CH
EX0 = NW * MAIN
GRP = 6
NGRP = CPT // GRP
NPAD = 10240
RPT = NPAD // NS
RB = 2000


@functools.cache
def _get_sc_kernel():
    mesh = plsc.VectorSubcoreMesh(core_axis_name="c", subcore_axis_name="s")

    @functools.partial(
        pl.kernel,
        mesh=mesh,
        compiler_params=pltpu.CompilerParams(use_tc_tiling_on_sc=False),
        out_type=[
            jax.ShapeDtypeStruct((NC, NPAD, C), jnp.float32),
            jax.ShapeDtypeStruct((NC, NPAD), jnp.float32),
        ],
        scratch_types=[
            pltpu.VMEM((MAIN + CH,), jnp.int32),
            pltpu.VMEM((MAIN + CH,), jnp.int32),
            pltpu.VMEM((CPT + 1, CH), jnp.int32),
            pltpu.VMEM((GRP, CH, C), jnp.float32),
            pltpu.VMEM((GRP, CH, C), jnp.float32),
            pltpu.VMEM((CH,), jnp.float32),
            pltpu.VMEM((RPT, C), jnp.float32),
            pltpu.VMEM((RPT,), jnp.float32),
            pltpu.VMEM_SHARED((NPAD, C), jnp.float32),
            pltpu.VMEM_SHARED((NPAD,), jnp.float32),
            pltpu.SemaphoreType.DMA,
            pltpu.SemaphoreType.DMA,
            pltpu.SemaphoreType.DMA,
            pltpu.SemaphoreType.DMA,
        ],
    )
    def _sc_edge_aggregate(ei_hbm, q_hbm, out_acc, out_deg,
                           src_v, dst_f, dst_v, rows_a, rows_b, ones_v,
                           zrows, zcol, acc_sh, deg_sh,
                           gsem_a, gsem_b, ssem_a, ssem_b):
        _sc_body(ei_hbm, q_hbm, out_acc, out_deg,
                 src_v, dst_f, dst_v, rows_a, rows_b, ones_v,
                 zrows, zcol, acc_sh, deg_sh,
                 gsem_a, gsem_b, ssem_a, ssem_b)

    return _sc_edge_aggregate


def _sc_body(ei_hbm, q_hbm, out_acc, out_deg,
             src_v, dst_f, dst_v, rows_a, rows_b, ones_v,
             zrows, zcol, acc_sh, deg_sh,
             gsem_a, gsem_b, ssem_a, ssem_b):
    c = lax.axis_index("c")
    s = lax.axis_index("s")
    w = c * NS + s

    zero16 = jnp.zeros((16,), jnp.float32)
    one16 = jnp.ones((16,), jnp.float32)

    pltpu.sync_copy(ei_hbm.at[0, pl.ds(w * MAIN, MAIN)],
                    src_v.at[pl.ds(0, MAIN)])
    pltpu.sync_copy(ei_hbm.at[1, pl.ds(w * MAIN, MAIN)],
                    dst_f.at[pl.ds(0, MAIN)])

    @pl.when(w < 4)
    def _stage_extra():
        pltpu.sync_copy(ei_hbm.at[0, pl.ds(EX0 + w * CH, CH)],
                        src_v.at[pl.ds(MAIN, CH)])
        pltpu.sync_copy(ei_hbm.at[1, pl.ds(EX0 + w * CH, CH)],
                        dst_f.at[pl.ds(MAIN, CH)])

    def dfill(j, carry):
        for k in range(CH // 16):
            dst_v[j, pl.ds(k * 16, 16)] = dst_f[pl.ds(j * CH + k * 16, 16)]
        return carry

    lax.fori_loop(0, CPT + 1, dfill, 0)

    def zfill_rows(i, carry):
        zrows[i, pl.ds(0, 16)] = zero16
        zrows[i, pl.ds(16, 16)] = zero16
        return carry

    lax.fori_loop(0, RPT, zfill_rows, 0)

    def zfill_col(i, carry):
        zcol[pl.ds(i * 16, 16)] = zero16
        return carry

    lax.fori_loop(0, RPT // 16, zfill_col, 0)

    for i in range(CH // 16):
        ones_v[pl.ds(i * 16, 16)] = one16

    pltpu.sync_copy(zrows, acc_sh.at[pl.ds(s * RPT, RPT)])
    pltpu.sync_copy(zcol, deg_sh.at[pl.ds(s * RPT, RPT)])
    plsc.subcore_barrier()

    def fire_gather(g, buf, sem):
        for i in range(GRP):
            pltpu.async_copy(
                q_hbm.at[src_v.at[pl.ds(g * (GRP * CH) + i * CH, CH)]],
                buf.at[i], sem)

    def drain_gather(g, buf, sem):
        for i in range(GRP):
            pltpu.make_async_copy(
                q_hbm.at[src_v.at[pl.ds(g * (GRP * CH) + i * CH, CH)]],
                buf.at[i], sem).wait()

    def fire_scatter(g, buf, sem):
        for i in range(GRP):
            pltpu.async_copy(buf.at[i], acc_sh.at[dst_v.at[g * GRP + i]],
                             sem, add=True)
            pltpu.async_copy(ones_v, deg_sh.at[dst_v.at[g * GRP + i]],
                             sem, add=True)

    def drain_scatter(g, buf, sem):
        for i in range(GRP):
            pltpu.make_async_copy(buf.at[i], acc_sh.at[dst_v.at[g * GRP + i]],
                                  sem).wait()
            pltpu.make_async_copy(ones_v, deg_sh.at[dst_v.at[g * GRP + i]],
                                  sem).wait()

    fire_gather(0, rows_a, gsem_a)

    def grp_pair(m, carry):
        g0 = m * 2
        fire_gather(g0 + 1, rows_b, gsem_b)
        drain_gather(g0, rows_a, gsem_a)
        fire_scatter(g0, rows_a, ssem_a)
        drain_gather(g0 + 1, rows_b, gsem_b)
        fire_scatter(g0 + 1, rows_b, ssem_b)
        drain_scatter(g0, rows_a, ssem_a)
        fire_gather(g0 + 2, rows_a, gsem_a)
        drain_scatter(g0 + 1, rows_b, ssem_b)
        return carry

    lax.fori_loop(0, (NGRP - 1) // 2, grp_pair, 0)

    last = NGRP - 1
    drain_gather(last, rows_a, gsem_a)
    fire_scatter(last, rows_a, ssem_a)
    drain_scatter(last, rows_a, ssem_a)

    @pl.when(w < 4)
    def _extra_chunk():
        pltpu.async_copy(q_hbm.at[src_v.at[pl.ds(MAIN, CH)]],
                         rows_a.at[0], gsem_a).wait()
        pltpu.async_copy(rows_a.at[0], acc_sh.at[dst_v.at[CPT]],
                         ssem_a, add=True).wait()
        pltpu.async_copy(ones_v, deg_sh.at[dst_v.at[CPT]],
                         ssem_a, add=True).wait()

    plsc.subcore_barrier()

    pltpu.sync_copy(acc_sh.at[pl.ds(s * RPT, RPT)], zrows)
    pltpu.sync_copy(zrows, out_acc.at[c, pl.ds(s * RPT, RPT)])
    pltpu.sync_copy(deg_sh.at[pl.ds(s * RPT, RPT)], zcol)
    pltpu.sync_copy(zcol, out_deg.at[c, pl.ds(s * RPT, RPT)])


def _precompute_tables(x, w_mlp, b_mlp):
    def body(x_ref, w_ref, b_ref, p_ref, q_ref):
        xb = x_ref[...]
        wt16 = w_ref[0:D, :].astype(jnp.bfloat16)
        wb16 = w_ref[D:2 * D, :].astype(jnp.bfloat16).astype(jnp.float32)
        qv = jnp.dot(xb, wb16,
                     preferred_element_type=jnp.float32,
                     precision=jax.lax.Precision.HIGHEST)
        p_ref[...] = (jnp.dot(xb.astype(jnp.bfloat16), wt16,
                              preferred_element_type=jnp.float32)
                      - qv + b_ref[...])
        q_ref[...] = qv
    return pl.pallas_call(
        body,
        grid=(N // RB,),
        in_specs=[
            pl.BlockSpec((RB, D), lambda i: (i, 0)),
            pl.BlockSpec((2 * D, C), lambda i: (0, 0)),
            pl.BlockSpec((1, C), lambda i: (0, 0)),
        ],
        out_specs=[
            pl.BlockSpec((RB, C), lambda i: (i, 0)),
            pl.BlockSpec((RB, C), lambda i: (i, 0)),
        ],
        out_shape=[
            jax.ShapeDtypeStruct((N, C), jnp.float32),
            jax.ShapeDtypeStruct((N, C), jnp.float32),
        ],
    )(x, w_mlp, b_mlp)


def _heads(p, acc, deg, gamma, beta, moving_mean, moving_var, w1, b1, w2, b2):
    def body(p_ref, acc_ref, deg_ref, g_ref, be_ref, mm_ref, v_ref,
             w1_ref, b1_ref, w2_ref, b2_ref, o_ref):
        degb = deg_ref[0] + deg_ref[1]
        h = degb * p_ref[...] + acc_ref[0] + acc_ref[1]
        hb = (g_ref[...] * (h - mm_ref[...])
              / jnp.sqrt(v_ref[...] + 1e-3) + be_ref[...])
        u = jnp.maximum(
            jnp.dot(hb.astype(jnp.bfloat16), w1_ref[...].astype(jnp.bfloat16),
                    preferred_element_type=jnp.float32) + b1_ref[...], 0.0)
        z = (jnp.dot(u.astype(jnp.bfloat16), w2_ref[...].astype(jnp.bfloat16),
                     preferred_element_type=jnp.float32) + b2_ref[...])
        o_ref[...] = jax.nn.sigmoid(z)
    return pl.pallas_call(
        body,
        grid=(N // RB,),
        in_specs=[
            pl.BlockSpec((RB, C), lambda i: (i, 0)),
            pl.BlockSpec((NC, RB, C), lambda i: (0, i, 0)),
            pl.BlockSpec((NC, RB, 1), lambda i: (0, i, 0)),
            pl.BlockSpec((1, C), lambda i: (0, 0)),
            pl.BlockSpec((1, C), lambda i: (0, 0)),
            pl.BlockSpec((1, C), lambda i: (0, 0)),
            pl.BlockSpec((1, C), lambda i: (0, 0)),
            pl.BlockSpec((C, 16), lambda i: (0, 0)),
            pl.BlockSpec((1, 16), lambda i: (0, 0)),
            pl.BlockSpec((16, 1), lambda i: (0, 0)),
            pl.BlockSpec((1, 1), lambda i: (0, 0)),
        ],
        out_specs=pl.BlockSpec((RB, 1), lambda i: (i, 0)),
        out_shape=jax.ShapeDtypeStruct((N, 1), jnp.float32),
    )(p, acc, deg, gamma, beta, moving_mean, moving_var, w1, b1, w2, b2)


def kernel(x, edge_index, W_mlp, b_mlp, gamma, beta, moving_mean,
           moving_var, W1, b1, W2, b2):
    g2 = gamma.reshape(1, C)
    be2 = beta.reshape(1, C)
    mm2 = moving_mean.reshape(1, C)
    mv2 = moving_var.reshape(1, C)

    p, q = _precompute_tables(x, W_mlp, b_mlp.reshape(1, C))

    acc, deg = _get_sc_kernel()(edge_index, q)

    return _heads(p, acc, deg.reshape(NC, NPAD, 1), g2, be2, mm2, mv2,
                  W1, b1.reshape(1, 16), W2, b2.reshape(1, 1))


# speedup vs baseline: 24.0538x; 1.0533x over previous
import functools

import jax
import jax.numpy as jnp
from jax import lax
from jax.experimental import pallas as pl
from jax.experimental.pallas import tpu as pltpu
from jax.experimental.pallas import tpu_sc as plsc

N = 10000
E = 320000
D = 128
C = 32

NC = 2
NS = 16
NW = NC * NS
CH = 128
CPT = 78
MAIN = CPT * CH
EX0 = NW * MAIN
GRP = 6
NGRP = CPT // GRP
NPAD = 10240
RPT = NPAD // NS
RB = 2000


@functools.cache
def _get_sc_kernel():
    mesh = plsc.VectorSubcoreMesh(core_axis_name="c", subcore_axis_name="s")

    @functools.partial(
        pl.kernel,
        mesh=mesh,
        compiler_params=pltpu.CompilerParams(use_tc_tiling_on_sc=False),
        out_type=[
            jax.ShapeDtypeStruct((NC, NPAD, C), jnp.float32),
            jax.ShapeDtypeStruct((NC, NPAD), jnp.float32),
        ],
        scratch_types=[
            pltpu.VMEM((MAIN + CH,), jnp.int32),
            pltpu.VMEM((MAIN + CH,), jnp.int32),
            pltpu.VMEM((CPT + 1, CH), jnp.int32),
            pltpu.VMEM((GRP, CH, C), jnp.float32),
            pltpu.VMEM((GRP, CH, C), jnp.float32),
            pltpu.VMEM((CH,), jnp.float32),
            pltpu.VMEM((RPT, C), jnp.float32),
            pltpu.VMEM((RPT,), jnp.float32),
            pltpu.VMEM_SHARED((NPAD, C), jnp.float32),
            pltpu.VMEM_SHARED((NPAD,), jnp.float32),
            pltpu.SemaphoreType.DMA,
            pltpu.SemaphoreType.DMA,
            pltpu.SemaphoreType.DMA,
            pltpu.SemaphoreType.DMA,
        ],
    )
    def _sc_edge_aggregate(ei_hbm, q_hbm, out_acc, out_deg,
                           src_v, dst_f, dst_v, rows_a, rows_b, ones_v,
                           zrows, zcol, acc_sh, deg_sh,
                           gsem_a, gsem_b, ssem_a, ssem_b):
        _sc_body(ei_hbm, q_hbm, out_acc, out_deg,
                 src_v, dst_f, dst_v, rows_a, rows_b, ones_v,
                 zrows, zcol, acc_sh, deg_sh,
                 gsem_a, gsem_b, ssem_a, ssem_b)

    return _sc_edge_aggregate


def _sc_body(ei_hbm, q_hbm, out_acc, out_deg,
             src_v, dst_f, dst_v, rows_a, rows_b, ones_v,
             zrows, zcol, acc_sh, deg_sh,
             gsem_a, gsem_b, ssem_a, ssem_b):
    c = lax.axis_index("c")
    s = lax.axis_index("s")
    w = c * NS + s

    zero16 = jnp.zeros((16,), jnp.float32)
    one16 = jnp.ones((16,), jnp.float32)

    pltpu.sync_copy(ei_hbm.at[0, pl.ds(w * MAIN, MAIN)],
                    src_v.at[pl.ds(0, MAIN)])
    pltpu.sync_copy(ei_hbm.at[1, pl.ds(w * MAIN, MAIN)],
                    dst_f.at[pl.ds(0, MAIN)])

    @pl.when(w < 4)
    def _stage_extra():
        pltpu.sync_copy(ei_hbm.at[0, pl.ds(EX0 + w * CH, CH)],
                        src_v.at[pl.ds(MAIN, CH)])
        pltpu.sync_copy(ei_hbm.at[1, pl.ds(EX0 + w * CH, CH)],
                        dst_f.at[pl.ds(MAIN, CH)])

    def dfill(j, carry):
        for k in range(CH // 16):
            dst_v[j, pl.ds(k * 16, 16)] = dst_f[pl.ds(j * CH + k * 16, 16)]
        return carry

    lax.fori_loop(0, CPT + 1, dfill, 0)

    def zfill_rows(i, carry):
        zrows[i, pl.ds(0, 16)] = zero16
        zrows[i, pl.ds(16, 16)] = zero16
        return carry

    lax.fori_loop(0, RPT, zfill_rows, 0)

    def zfill_col(i, carry):
        zcol[pl.ds(i * 16, 16)] = zero16
        return carry

    lax.fori_loop(0, RPT // 16, zfill_col, 0)

    for i in range(CH // 16):
        ones_v[pl.ds(i * 16, 16)] = one16

    pltpu.sync_copy(zrows, acc_sh.at[pl.ds(s * RPT, RPT)])
    pltpu.sync_copy(zcol, deg_sh.at[pl.ds(s * RPT, RPT)])
    plsc.subcore_barrier()

    def fire_gather(g, buf, sem):
        for i in range(GRP):
            pltpu.async_copy(
                q_hbm.at[src_v.at[pl.ds(g * (GRP * CH) + i * CH, CH)]],
                buf.at[i], sem)

    def drain_gather(g, buf, sem):
        for i in range(GRP):
            pltpu.make_async_copy(
                q_hbm.at[src_v.at[pl.ds(g * (GRP * CH) + i * CH, CH)]],
                buf.at[i], sem).wait()

    def fire_scatter(g, buf, sem):
        for i in range(GRP):
            pltpu.async_copy(buf.at[i], acc_sh.at[dst_v.at[g * GRP + i]],
                             sem, add=True)
            pltpu.async_copy(ones_v, deg_sh.at[dst_v.at[g * GRP + i]],
                             sem, add=True)

    def drain_scatter(g, buf, sem):
        for i in range(GRP):
            pltpu.make_async_copy(buf.at[i], acc_sh.at[dst_v.at[g * GRP + i]],
                                  sem).wait()
            pltpu.make_async_copy(ones_v, deg_sh.at[dst_v.at[g * GRP + i]],
                                  sem).wait()

    fire_gather(0, rows_a, gsem_a)

    def grp_pair(m, carry):
        g0 = m * 2
        fire_gather(g0 + 1, rows_b, gsem_b)
        drain_gather(g0, rows_a, gsem_a)
        fire_scatter(g0, rows_a, ssem_a)
        drain_gather(g0 + 1, rows_b, gsem_b)
        fire_scatter(g0 + 1, rows_b, ssem_b)
        drain_scatter(g0, rows_a, ssem_a)
        fire_gather(g0 + 2, rows_a, gsem_a)
        drain_scatter(g0 + 1, rows_b, ssem_b)
        return carry

    lax.fori_loop(0, (NGRP - 1) // 2, grp_pair, 0)

    last = NGRP - 1
    drain_gather(last, rows_a, gsem_a)
    fire_scatter(last, rows_a, ssem_a)
    drain_scatter(last, rows_a, ssem_a)

    @pl.when(w < 4)
    def _extra_chunk():
        pltpu.async_copy(q_hbm.at[src_v.at[pl.ds(MAIN, CH)]],
                         rows_a.at[0], gsem_a).wait()
        pltpu.async_copy(rows_a.at[0], acc_sh.at[dst_v.at[CPT]],
                         ssem_a, add=True).wait()
        pltpu.async_copy(ones_v, deg_sh.at[dst_v.at[CPT]],
                         ssem_a, add=True).wait()

    plsc.subcore_barrier()

    pltpu.sync_copy(acc_sh.at[pl.ds(s * RPT, RPT)], zrows)
    pltpu.sync_copy(zrows, out_acc.at[c, pl.ds(s * RPT, RPT)])
    pltpu.sync_copy(deg_sh.at[pl.ds(s * RPT, RPT)], zcol)
    pltpu.sync_copy(zcol, out_deg.at[c, pl.ds(s * RPT, RPT)])


def _precompute_tables(x, w_mlp, b_mlp):
    def body(x_ref, w_ref, b_ref, p_ref, q_ref):
        xb = x_ref[...]
        wt16 = w_ref[0:D, :].astype(jnp.bfloat16)
        wb16 = w_ref[D:2 * D, :].astype(jnp.bfloat16).astype(jnp.float32)
        qv = jnp.dot(xb, wb16,
                     preferred_element_type=jnp.float32,
                     precision=jax.lax.Precision.HIGHEST)
        p_ref[...] = (jnp.dot(xb.astype(jnp.bfloat16), wt16,
                              preferred_element_type=jnp.float32)
                      - qv + b_ref[...])
        q_ref[...] = qv
    return pl.pallas_call(
        body,
        grid=(N // RB,),
        in_specs=[
            pl.BlockSpec((RB, D), lambda i: (i, 0)),
            pl.BlockSpec((2 * D, C), lambda i: (0, 0)),
            pl.BlockSpec((1, C), lambda i: (0, 0)),
        ],
        out_specs=[
            pl.BlockSpec((RB, C), lambda i: (i, 0)),
            pl.BlockSpec((RB, C), lambda i: (i, 0)),
        ],
        out_shape=[
            jax.ShapeDtypeStruct((N, C), jnp.float32),
            jax.ShapeDtypeStruct((N, C), jnp.float32),
        ],
    )(x, w_mlp, b_mlp)


def _heads(p, acc, deg, gamma, beta, moving_mean, moving_var, w1, b1, w2, b2):
    def body(p_ref, acc_ref, deg_ref, g_ref, be_ref, mm_ref, v_ref,
             w1_ref, b1_ref, w2_ref, b2_ref, o_ref):
        h = deg_ref[...] * p_ref[...] + acc_ref[0] + acc_ref[1]
        hb = (g_ref[...] * (h - mm_ref[...])
              / jnp.sqrt(v_ref[...] + 1e-3) + be_ref[...])
        u = jnp.maximum(
            jnp.dot(hb.astype(jnp.bfloat16), w1_ref[...].astype(jnp.bfloat16),
                    preferred_element_type=jnp.float32) + b1_ref[...], 0.0)
        z = (jnp.dot(u.astype(jnp.bfloat16), w2_ref[...].astype(jnp.bfloat16),
                     preferred_element_type=jnp.float32) + b2_ref[...])
        o_ref[...] = jax.nn.sigmoid(z)
    return pl.pallas_call(
        body,
        grid=(N // RB,),
        in_specs=[
            pl.BlockSpec((RB, C), lambda i: (i, 0)),
            pl.BlockSpec((NC, RB, C), lambda i: (0, i, 0)),
            pl.BlockSpec((RB, C), lambda i: (i, 0)),
            pl.BlockSpec((1, C), lambda i: (0, 0)),
            pl.BlockSpec((1, C), lambda i: (0, 0)),
            pl.BlockSpec((1, C), lambda i: (0, 0)),
            pl.BlockSpec((1, C), lambda i: (0, 0)),
            pl.BlockSpec((C, 16), lambda i: (0, 0)),
            pl.BlockSpec((1, 16), lambda i: (0, 0)),
            pl.BlockSpec((16, 1), lambda i: (0, 0)),
            pl.BlockSpec((1, 1), lambda i: (0, 0)),
        ],
        out_specs=pl.BlockSpec((RB, 1), lambda i: (i, 0)),
        out_shape=jax.ShapeDtypeStruct((N, 1), jnp.float32),
    )(p, acc, deg, gamma, beta, moving_mean, moving_var, w1, b1, w2, b2)


def kernel(x, edge_index, W_mlp, b_mlp, gamma, beta, moving_mean,
           moving_var, W1, b1, W2, b2):
    g2 = gamma.reshape(1, C)
    be2 = beta.reshape(1, C)
    mm2 = moving_mean.reshape(1, C)
    mv2 = moving_var.reshape(1, C)

    p, q = _precompute_tables(x, W_mlp, b_mlp.reshape(1, C))

    acc, deg = _get_sc_kernel()(edge_index, q)

    degb = jnp.broadcast_to((deg[0] + deg[1])[:, None], (NPAD, C))
    return _heads(p, acc, degb, g2, be2, mm2, mv2,
                  W1, b1.reshape(1, 16), W2, b2.reshape(1, 1))


# speedup vs baseline: 24.1381x; 1.0035x over previous
import functools

import jax
import jax.numpy as jnp
from jax import lax
from jax.experimental import pallas as pl
from jax.experimental.pallas import tpu as pltpu
from jax.experimental.pallas import tpu_sc as plsc

N = 10000
E = 320000
D = 128
C = 32

NC = 2
NS = 16
NW = NC * NS
CH = 128
CPT = 78
MAIN = CPT * CH
EX0 = NW * MAIN
GRP = 6
NGRP = CPT // GRP
NPAD = 10240
RPT = NPAD // NS
RB = 2000


@functools.cache
def _get_sc_kernel():
    mesh = plsc.VectorSubcoreMesh(core_axis_name="c", subcore_axis_name="s")

    @functools.partial(
        pl.kernel,
        mesh=mesh,
        compiler_params=pltpu.CompilerParams(use_tc_tiling_on_sc=False),
        out_type=[
            jax.ShapeDtypeStruct((NC, NPAD, C), jnp.float32),
            jax.ShapeDtypeStruct((NC, NPAD), jnp.float32),
        ],
        scratch_types=[
            pltpu.VMEM((MAIN + CH,), jnp.int32),
            pltpu.VMEM((MAIN + CH,), jnp.int32),
            pltpu.VMEM((CPT + 1, CH), jnp.int32),
            pltpu.VMEM((GRP, CH, C), jnp.float32),
            pltpu.VMEM((GRP, CH, C), jnp.float32),
            pltpu.VMEM((CH,), jnp.float32),
            pltpu.VMEM((RPT, C), jnp.float32),
            pltpu.VMEM((RPT,), jnp.float32),
            pltpu.VMEM_SHARED((NPAD, C), jnp.float32),
            pltpu.VMEM_SHARED((NPAD,), jnp.float32),
            pltpu.SemaphoreType.DMA,
            pltpu.SemaphoreType.DMA,
            pltpu.SemaphoreType.DMA,
            pltpu.SemaphoreType.DMA,
        ],
    )
    def _sc_edge_aggregate(ei_hbm, q_hbm, out_acc, out_deg,
                           src_v, dst_f, dst_v, rows_a, rows_b, ones_v,
                           zrows, zcol, acc_sh, deg_sh,
                           gsem_a, gsem_b, ssem_a, ssem_b):
        _sc_body(ei_hbm, q_hbm, out_acc, out_deg,
                 src_v, dst_f, dst_v, rows_a, rows_b, ones_v,
                 zrows, zcol, acc_sh, deg_sh,
                 gsem_a, gsem_b, ssem_a, ssem_b)

    return _sc_edge_aggregate


def _sc_body(ei_hbm, q_hbm, out_acc, out_deg,
             src_v, dst_f, dst_v, rows_a, rows_b, ones_v,
             zrows, zcol, acc_sh, deg_sh,
             gsem_a, gsem_b, ssem_a, ssem_b):
    c = lax.axis_index("c")
    s = lax.axis_index("s")
    w = c * NS + s

    zero16 = jnp.zeros((16,), jnp.float32)
    one16 = jnp.ones((16,), jnp.float32)

    pltpu.sync_copy(ei_hbm.at[pl.ds(w * MAIN, MAIN)],
                    src_v.at[pl.ds(0, MAIN)])
    pltpu.sync_copy(ei_hbm.at[pl.ds(E + w * MAIN, MAIN)],
                    dst_f.at[pl.ds(0, MAIN)])

    @pl.when(w < 4)
    def _stage_extra():
        pltpu.sync_copy(ei_hbm.at[pl.ds(EX0 + w * CH, CH)],
                        src_v.at[pl.ds(MAIN, CH)])
        pltpu.sync_copy(ei_hbm.at[pl.ds(E + EX0 + w * CH, CH)],
                        dst_f.at[pl.ds(MAIN, CH)])

    def dfill(j, carry):
        for k in range(CH // 16):
            dst_v[j, pl.ds(k * 16, 16)] = dst_f[pl.ds(j * CH + k * 16, 16)]
        return carry

    lax.fori_loop(0, CPT + 1, dfill, 0)

    def zfill_rows(i, carry):
        zrows[i, pl.ds(0, 16)] = zero16
        zrows[i, pl.ds(16, 16)] = zero16
        return carry

    lax.fori_loop(0, RPT, zfill_rows, 0)

    def zfill_col(i, carry):
        zcol[pl.ds(i * 16, 16)] = zero16
        return carry

    lax.fori_loop(0, RPT // 16, zfill_col, 0)

    for i in range(CH // 16):
        ones_v[pl.ds(i * 16, 16)] = one16

    pltpu.sync_copy(zrows, acc_sh.at[pl.ds(s * RPT, RPT)])
    pltpu.sync_copy(zcol, deg_sh.at[pl.ds(s * RPT, RPT)])
    plsc.subcore_barrier()

    def fire_gather(g, buf, sem):
        for i in range(GRP):
            pltpu.async_copy(
                q_hbm.at[src_v.at[pl.ds(g * (GRP * CH) + i * CH, CH)]],
                buf.at[i], sem)

    def drain_gather(g, buf, sem):
        for i in range(GRP):
            pltpu.make_async_copy(
                q_hbm.at[src_v.at[pl.ds(g * (GRP * CH) + i * CH, CH)]],
                buf.at[i], sem).wait()

    def fire_scatter(g, buf, sem):
        for i in range(GRP):
            pltpu.async_copy(buf.at[i], acc_sh.at[dst_v.at[g * GRP + i]],
                             sem, add=True)
            pltpu.async_copy(ones_v, deg_sh.at[dst_v.at[g * GRP + i]],
                             sem, add=True)

    def drain_scatter(g, buf, sem):
        for i in range(GRP):
            pltpu.make_async_copy(buf.at[i], acc_sh.at[dst_v.at[g * GRP + i]],
                                  sem).wait()
            pltpu.make_async_copy(ones_v, deg_sh.at[dst_v.at[g * GRP + i]],
                                  sem).wait()

    fire_gather(0, rows_a, gsem_a)

    def grp_pair(m, carry):
        g0 = m * 2
        fire_gather(g0 + 1, rows_b, gsem_b)
        drain_gather(g0, rows_a, gsem_a)
        fire_scatter(g0, rows_a, ssem_a)
        drain_gather(g0 + 1, rows_b, gsem_b)
        fire_scatter(g0 + 1, rows_b, ssem_b)
        drain_scatter(g0, rows_a, ssem_a)
        fire_gather(g0 + 2, rows_a, gsem_a)
        drain_scatter(g0 + 1, rows_b, ssem_b)
        return carry

    lax.fori_loop(0, (NGRP - 1) // 2, grp_pair, 0)

    last = NGRP - 1
    drain_gather(last, rows_a, gsem_a)
    fire_scatter(last, rows_a, ssem_a)
    drain_scatter(last, rows_a, ssem_a)

    @pl.when(w < 4)
    def _extra_chunk():
        pltpu.async_copy(q_hbm.at[src_v.at[pl.ds(MAIN, CH)]],
                         rows_a.at[0], gsem_a).wait()
        pltpu.async_copy(rows_a.at[0], acc_sh.at[dst_v.at[CPT]],
                         ssem_a, add=True).wait()
        pltpu.async_copy(ones_v, deg_sh.at[dst_v.at[CPT]],
                         ssem_a, add=True).wait()

    plsc.subcore_barrier()

    pltpu.sync_copy(acc_sh.at[pl.ds(s * RPT, RPT)], zrows)
    pltpu.sync_copy(zrows, out_acc.at[c, pl.ds(s * RPT, RPT)])
    pltpu.sync_copy(deg_sh.at[pl.ds(s * RPT, RPT)], zcol)
    pltpu.sync_copy(zcol, out_deg.at[c, pl.ds(s * RPT, RPT)])


def _precompute_tables(x, w_mlp, b_mlp):
    def body(x_ref, w_ref, b_ref, p_ref, q_ref):
        xb = x_ref[...]
        wt16 = w_ref[0:D, :].astype(jnp.bfloat16)
        wb16 = w_ref[D:2 * D, :].astype(jnp.bfloat16).astype(jnp.float32)
        qv = jnp.dot(xb, wb16,
                     preferred_element_type=jnp.float32,
                     precision=jax.lax.Precision.HIGHEST)
        p_ref[...] = (jnp.dot(xb.astype(jnp.bfloat16), wt16,
                              preferred_element_type=jnp.float32)
                      - qv + b_ref[...])
        q_ref[...] = qv
    return pl.pallas_call(
        body,
        grid=(N // RB,),
        in_specs=[
            pl.BlockSpec((RB, D), lambda i: (i, 0)),
            pl.BlockSpec((2 * D, C), lambda i: (0, 0)),
            pl.BlockSpec((1, C), lambda i: (0, 0)),
        ],
        out_specs=[
            pl.BlockSpec((RB, C), lambda i: (i, 0)),
            pl.BlockSpec((RB, C), lambda i: (i, 0)),
        ],
        out_shape=[
            jax.ShapeDtypeStruct((N, C), jnp.float32),
            jax.ShapeDtypeStruct((N, C), jnp.float32),
        ],
    )(x, w_mlp, b_mlp)


def _heads(p, acc, deg, gamma, beta, moving_mean, moving_var, w1, b1, w2, b2):
    def body(p_ref, acc_ref, deg_ref, g_ref, be_ref, mm_ref, v_ref,
             w1_ref, b1_ref, w2_ref, b2_ref, o_ref):
        h = deg_ref[...] * p_ref[...] + acc_ref[0] + acc_ref[1]
        hb = (g_ref[...] * (h - mm_ref[...])
              / jnp.sqrt(v_ref[...] + 1e-3) + be_ref[...])
        u = jnp.maximum(
            jnp.dot(hb.astype(jnp.bfloat16), w1_ref[...].astype(jnp.bfloat16),
                    preferred_element_type=jnp.float32) + b1_ref[...], 0.0)
        z = (jnp.dot(u.astype(jnp.bfloat16), w2_ref[...].astype(jnp.bfloat16),
                     preferred_element_type=jnp.float32) + b2_ref[...])
        o_ref[...] = jax.nn.sigmoid(z)
    return pl.pallas_call(
        body,
        grid=(N // RB,),
        in_specs=[
            pl.BlockSpec((RB, C), lambda i: (i, 0)),
            pl.BlockSpec((NC, RB, C), lambda i: (0, i, 0)),
            pl.BlockSpec((RB, C), lambda i: (i, 0)),
            pl.BlockSpec((1, C), lambda i: (0, 0)),
            pl.BlockSpec((1, C), lambda i: (0, 0)),
            pl.BlockSpec((1, C), lambda i: (0, 0)),
            pl.BlockSpec((1, C), lambda i: (0, 0)),
            pl.BlockSpec((C, 16), lambda i: (0, 0)),
            pl.BlockSpec((1, 16), lambda i: (0, 0)),
            pl.BlockSpec((16, 1), lambda i: (0, 0)),
            pl.BlockSpec((1, 1), lambda i: (0, 0)),
        ],
        out_specs=pl.BlockSpec((RB, 1), lambda i: (i, 0)),
        out_shape=jax.ShapeDtypeStruct((N, 1), jnp.float32),
    )(p, acc, deg, gamma, beta, moving_mean, moving_var, w1, b1, w2, b2)


def kernel(x, edge_index, W_mlp, b_mlp, gamma, beta, moving_mean,
           moving_var, W1, b1, W2, b2):
    g2 = gamma.reshape(1, C)
    be2 = beta.reshape(1, C)
    mm2 = moving_mean.reshape(1, C)
    mv2 = moving_var.reshape(1, C)

    p, q = _precompute_tables(x, W_mlp, b_mlp.reshape(1, C))

    acc, deg = _get_sc_kernel()(edge_index.reshape(2 * E), q)

    degb = jnp.broadcast_to((deg[0] + deg[1])[:, None], (NPAD, C))
    return _heads(p, acc, degb, g2, be2, mm2, mv2,
                  W1, b1.reshape(1, 16), W2, b2.reshape(1, 1))
